# baseline (speedup 1.0000x reference)
import functools

import jax
import jax.numpy as jnp
from jax import lax
from jax.experimental import pallas as pl
from jax.experimental.pallas import tpu as pltpu
from jax.experimental.pallas import tpu_sc as plsc

EPS = 1e-9
CAPACITY_FACTOR = 1.25
MIN_CAPACITY = 4

_NW = 32
_WTOK = 16


def _phase1_body(x_ref, w_ref, meta_ref, *, cap, num_gates, group_size):
    xb = x_ref[0]
    w = w_ref[...]
    logits = lax.dot_general(
        xb, w, (((1,), (1,)), ((), ())), preferred_element_type=jnp.float32
    )
    m = jnp.max(logits, axis=-1, keepdims=True)
    ex = jnp.exp(logits - m)
    raw = ex / jnp.sum(ex, axis=-1, keepdims=True)

    lane = lax.broadcasted_iota(jnp.int32, (group_size, num_gates), 1).astype(
        jnp.float32
    )
    g1 = jnp.max(raw, axis=-1, keepdims=True)
    i1 = jnp.min(jnp.where(raw >= g1, lane, jnp.float32(1e9)), axis=-1, keepdims=True)
    mask1 = (lane == i1).astype(jnp.float32)
    wo = raw * (1.0 - mask1)
    g2 = jnp.max(wo, axis=-1, keepdims=True)
    i2 = jnp.min(jnp.where(wo >= g2, lane, jnp.float32(1e9)), axis=-1, keepdims=True)
    mask2 = (lane == i2).astype(jnp.float32)

    denom = g1 + g2 + EPS
    g1n = g1 / denom
    g2n = g2 / denom

    proxy_m = jnp.mean(raw, axis=0, keepdims=True)
    dens1 = jnp.mean(mask1, axis=0, keepdims=True)
    partial = jnp.sum(proxy_m * dens1)

    r = lax.broadcasted_iota(jnp.int32, (group_size, group_size), 0)
    c = lax.broadcasted_iota(jnp.int32, (group_size, group_size), 1)
    lt = (r > c).astype(jnp.bfloat16)
    pos1 = jnp.dot(lt, mask1.astype(jnp.bfloat16),
                   preferred_element_type=jnp.float32)
    pos1_tok = jnp.sum(pos1 * mask1, axis=-1, keepdims=True)
    keep1 = (pos1_tok < cap).astype(jnp.float32)
    cnt1 = jnp.sum(mask1 * keep1, axis=0, keepdims=True)
    pos2 = jnp.dot(lt, mask2.astype(jnp.bfloat16),
                   preferred_element_type=jnp.float32) + cnt1
    pos2_tok = jnp.sum(pos2 * mask2, axis=-1, keepdims=True)
    keep2 = (pos2_tok < cap).astype(jnp.float32)

    val1 = g1n * keep1
    val2 = g2n * keep2
    p1c = jnp.minimum(pos1_tok, cap - 1.0)
    p2c = jnp.minimum(pos2_tok, cap - 1.0)
    losscol = jnp.zeros((group_size, 1), jnp.float32) + partial
    pad = jnp.zeros((group_size, 1), jnp.float32)
    meta_ref[0] = jnp.concatenate(
        [i1, p1c, val1, i2, p2c, val2, losscol, pad], axis=1
    )


def _make_sc_phase2(b, s, e, cap):
    tok_per_tile = (b * s) // _NW
    wt = 2
    nwin = tok_per_tile // wt
    zoffs = list(range(0, cap - 15, 16))
    if zoffs[-1] != cap - 16:
        zoffs.append(cap - 16)
    mesh = plsc.VectorSubcoreMesh(core_axis_name="c", subcore_axis_name="s")

    @functools.partial(
        pl.kernel,
        out_type=(
            jax.ShapeDtypeStruct((b, s, e, cap), jnp.float32),
            jax.ShapeDtypeStruct((b, s, e, cap), jnp.float32),
        ),
        mesh=mesh,
        scratch_types=[
            pltpu.VMEM((8 * tok_per_tile,), jnp.float32),
            pltpu.VMEM((wt, e, cap), jnp.float32),
            pltpu.VMEM((wt, e, cap), jnp.float32),
            pltpu.VMEM((wt, e, cap), jnp.float32),
            pltpu.VMEM((wt, e, cap), jnp.float32),
            pltpu.SemaphoreType.DMA,
            pltpu.SemaphoreType.DMA,
            pltpu.SemaphoreType.DMA,
        ],
        compiler_params=pltpu.CompilerParams(needs_layout_passes=False),
    )
    def sc_phase2(meta_hbm, comb_hbm, disp_hbm, mbuf, ca, cb, da, db,
                  sem0, sem1, msem):
        wid = lax.axis_index("s") * 2 + lax.axis_index("c")
        tok0 = wid * tok_per_tile
        g = tok0 // s
        t0 = tok0 % s

        mcp = pltpu.async_copy(
            meta_hbm.at[pl.ds(tok0 * 8, 8 * tok_per_tile)], mbuf, msem
        )

        zeros16 = jnp.zeros((16,), jnp.float32)
        for buf in (ca, cb, da, db):
            def zrow(i, _, _buf=buf):
                def zexp(j, _2, _i=i, _buf2=_buf):
                    for off in zoffs:
                        _buf2[_i, j, pl.ds(off, 16)] = zeros16
                    return 0
                return lax.fori_loop(0, e, zexp, 0)

            lax.fori_loop(0, wt, zrow, 0)

        mcp.wait()

        lane = lax.iota(jnp.int32, 16)
        tokin = jnp.bitwise_and(lane, 1)
        slotc = jnp.where(jnp.bitwise_and(lane, 2) > 0, 3, 0)
        act = lane < 4
        comb_slabs = (ca, cb)
        disp_slabs = (da, db)
        sems = (sem0, sem1)
        pend = [None, None]
        for w in range(nwin):
            p = w % 2
            if pend[p] is not None:
                h1, h2, oe, op = pend[p]
                h1.wait()
                h2.wait()
                plsc.store_scatter(comb_slabs[p], [tokin, oe, op], zeros16,
                                   mask=act)
                plsc.store_scatter(disp_slabs[p], [tokin, oe, op], zeros16,
                                   mask=act)
            mrow = (w * wt + tokin) * 8 + slotc
            ei = plsc.load_gather(mbuf, [mrow]).astype(jnp.int32)
            pi = plsc.load_gather(mbuf, [mrow + 1]).astype(jnp.int32)
            valf = plsc.load_gather(mbuf, [mrow + 2])
            plsc.store_scatter(comb_slabs[p], [tokin, ei, pi], valf, mask=act)
            plsc.store_scatter(
                disp_slabs[p], [tokin, ei, pi],
                jnp.where(valf > 0.0, 1.0, 0.0).astype(jnp.float32),
                mask=act,
            )
            h1 = pltpu.async_copy(
                comb_slabs[p], comb_hbm.at[g, pl.ds(t0 + w * wt, wt)], sems[p]
            )
            h2 = pltpu.async_copy(
                disp_slabs[p], disp_hbm.at[g, pl.ds(t0 + w * wt, wt)], sems[p]
            )
            pend[p] = (h1, h2, ei, pi)
        for p in (0, 1):
            pend[p][0].wait()
            pend[p][1].wait()

    return sc_phase2


def kernel(x, W):
    b, s, d = x.shape
    e = W.shape[0]
    cap = max(min(s, int(s * CAPACITY_FACTOR / e)), MIN_CAPACITY)

    meta = pl.pallas_call(
        functools.partial(
            _phase1_body, cap=float(cap), num_gates=e, group_size=s
        ),
        grid=(b,),
        in_specs=[
            pl.BlockSpec((1, s, d), lambda i: (i, 0, 0)),
            pl.BlockSpec((e, d), lambda i: (0, 0)),
        ],
        out_specs=pl.BlockSpec((1, s, 8), lambda i: (i, 0, 0)),
        out_shape=jax.ShapeDtypeStruct((b, s, 8), jnp.float32),
    )(x, W)

    meta_flat = meta.reshape(b * s * 8)
    combine, dispatch = _make_sc_phase2(b, s, e, cap)(meta_flat)

    loss = jnp.sum(meta[:, 0, 6]) * (float(e) / float(b))
    return (dispatch, combine, loss)


# speedup vs baseline: 2.2128x; 2.2128x over previous
import functools

import jax
import jax.numpy as jnp
from jax import lax
from jax.experimental import pallas as pl
from jax.experimental.pallas import tpu as pltpu
from jax.experimental.pallas import tpu_sc as plsc

EPS = 1e-9
CAPACITY_FACTOR = 1.25
MIN_CAPACITY = 4

_NW = 32


def _phase1_body(x_ref, w_ref, meta_ref, *, cap, num_gates, group_size):
    xb = x_ref[0]
    w = w_ref[...]
    logits = lax.dot_general(
        xb, w, (((1,), (1,)), ((), ())), preferred_element_type=jnp.float32
    )
    m = jnp.max(logits, axis=-1, keepdims=True)
    ex = jnp.exp(logits - m)
    raw = ex / jnp.sum(ex, axis=-1, keepdims=True)

    lane = lax.broadcasted_iota(jnp.int32, (group_size, num_gates), 1).astype(
        jnp.float32
    )
    g1 = jnp.max(raw, axis=-1, keepdims=True)
    i1 = jnp.min(jnp.where(raw >= g1, lane, jnp.float32(1e9)), axis=-1, keepdims=True)
    mask1 = (lane == i1).astype(jnp.float32)
    wo = raw * (1.0 - mask1)
    g2 = jnp.max(wo, axis=-1, keepdims=True)
    i2 = jnp.min(jnp.where(wo >= g2, lane, jnp.float32(1e9)), axis=-1, keepdims=True)
    mask2 = (lane == i2).astype(jnp.float32)

    denom = g1 + g2 + EPS
    g1n = g1 / denom
    g2n = g2 / denom

    proxy_m = jnp.mean(raw, axis=0, keepdims=True)
    dens1 = jnp.mean(mask1, axis=0, keepdims=True)
    partial = jnp.sum(proxy_m * dens1)

    r = lax.broadcasted_iota(jnp.int32, (group_size, group_size), 0)
    c = lax.broadcasted_iota(jnp.int32, (group_size, group_size), 1)
    lt = (r > c).astype(jnp.bfloat16)
    pos1 = jnp.dot(lt, mask1.astype(jnp.bfloat16),
                   preferred_element_type=jnp.float32)
    pos1_tok = jnp.sum(pos1 * mask1, axis=-1, keepdims=True)
    keep1 = (pos1_tok < cap).astype(jnp.float32)
    cnt1 = jnp.sum(mask1 * keep1, axis=0, keepdims=True)
    pos2 = jnp.dot(lt, mask2.astype(jnp.bfloat16),
                   preferred_element_type=jnp.float32) + cnt1
    pos2_tok = jnp.sum(pos2 * mask2, axis=-1, keepdims=True)
    keep2 = (pos2_tok < cap).astype(jnp.float32)

    val1 = g1n * keep1
    val2 = g2n * keep2
    p1c = jnp.minimum(pos1_tok, cap - 1.0)
    p2c = jnp.minimum(pos2_tok, cap - 1.0)
    losscol = jnp.zeros((group_size, 1), jnp.float32) + partial
    pad = jnp.zeros((group_size, 1), jnp.float32)
    meta_ref[0] = jnp.concatenate(
        [i1, p1c, val1, i2, p2c, val2, losscol, pad], axis=1
    )


def _make_sc_writer(b, s, e, cap, dispatch_mode):
    num_cols = e * cap
    tok_per_tile = (b * s) // _NW
    wt = 8
    nwin = tok_per_tile // wt
    mesh = plsc.VectorSubcoreMesh(core_axis_name="c", subcore_axis_name="s")

    @functools.partial(
        pl.kernel,
        out_type=jax.ShapeDtypeStruct((b, s, num_cols), jnp.float32),
        mesh=mesh,
        scratch_types=[
            pltpu.VMEM((8 * tok_per_tile,), jnp.float32),
            pltpu.VMEM((wt, num_cols), jnp.float32),
            pltpu.VMEM((wt, num_cols), jnp.float32),
            pltpu.SemaphoreType.DMA,
            pltpu.SemaphoreType.DMA,
            pltpu.SemaphoreType.DMA,
        ],
        compiler_params=pltpu.CompilerParams(needs_layout_passes=False),
    )
    def sc_writer(meta_hbm, out_hbm, mbuf, sa, sb, sem0, sem1, msem):
        wid = lax.axis_index("s") * 2 + lax.axis_index("c")
        tok0 = wid * tok_per_tile
        g = tok0 // s
        t0 = tok0 % s

        mcp = pltpu.async_copy(
            meta_hbm.at[pl.ds(tok0 * 8, 8 * tok_per_tile)], mbuf, msem
        )

        zeros16 = jnp.zeros((16,), jnp.float32)
        for buf in (sa, sb):
            def zrow(i, _, _buf=buf):
                def zcol(j, _2, _i=i, _buf2=_buf):
                    _buf2[_i, pl.ds(j * 16, 16)] = zeros16
                    return 0
                return lax.fori_loop(0, num_cols // 16, zcol, 0)

            lax.fori_loop(0, wt, zrow, 0)

        mcp.wait()

        lane = lax.iota(jnp.int32, 16)
        tokin = jnp.bitwise_and(lane, 7)
        slotc = jnp.where(lane >= 8, 3, 0)
        slabs = (sa, sb)
        sems = (sem0, sem1)
        pend = [None, None]
        for w in range(nwin):
            p = w % 2
            if pend[p] is not None:
                h, oc = pend[p]
                h.wait()
                plsc.store_scatter(slabs[p], [tokin, oc], zeros16)
            mrow = (w * wt + tokin) * 8 + slotc
            ei = plsc.load_gather(mbuf, [mrow]).astype(jnp.int32)
            pi = plsc.load_gather(mbuf, [mrow + 1]).astype(jnp.int32)
            valf = plsc.load_gather(mbuf, [mrow + 2])
            coli = ei * cap + pi
            if dispatch_mode:
                vals = jnp.where(valf > 0.0, 1.0, 0.0).astype(jnp.float32)
            else:
                vals = valf
            plsc.store_scatter(slabs[p], [tokin, coli], vals)
            h = pltpu.async_copy(
                slabs[p], out_hbm.at[g, pl.ds(t0 + w * wt, wt)], sems[p]
            )
            pend[p] = (h, coli)
        for p in (0, 1):
            pend[p][0].wait()

    return sc_writer


def kernel(x, W):
    b, s, d = x.shape
    e = W.shape[0]
    cap = max(min(s, int(s * CAPACITY_FACTOR / e)), MIN_CAPACITY)

    meta = pl.pallas_call(
        functools.partial(
            _phase1_body, cap=float(cap), num_gates=e, group_size=s
        ),
        grid=(b,),
        in_specs=[
            pl.BlockSpec((1, s, d), lambda i: (i, 0, 0)),
            pl.BlockSpec((e, d), lambda i: (0, 0)),
        ],
        out_specs=pl.BlockSpec((1, s, 8), lambda i: (i, 0, 0)),
        out_shape=jax.ShapeDtypeStruct((b, s, 8), jnp.float32),
    )(x, W)

    meta_flat = meta.reshape(b * s * 8)
    comb3 = _make_sc_writer(b, s, e, cap, False)(meta_flat)
    disp3 = _make_sc_writer(b, s, e, cap, True)(meta_flat)

    dispatch = disp3.reshape(b, s, e, cap)
    combine = comb3.reshape(b, s, e, cap)
    loss = jnp.sum(meta[:, 0, 6]) * (float(e) / float(b))
    return (dispatch, combine, loss)


# speedup vs baseline: 2.3776x; 1.0744x over previous
import functools

import jax
import jax.numpy as jnp
from jax import lax
from jax.experimental import pallas as pl

EPS = 1e-9
CAPACITY_FACTOR = 1.25
MIN_CAPACITY = 4


def _phase1_body(x_ref, w_ref, meta_ref, *, cap, num_gates, group_size):
    xb = x_ref[0]
    w = w_ref[...]
    logits = lax.dot_general(
        xb, w, (((1,), (1,)), ((), ())), preferred_element_type=jnp.float32
    )
    m = jnp.max(logits, axis=-1, keepdims=True)
    ex = jnp.exp(logits - m)
    raw = ex / jnp.sum(ex, axis=-1, keepdims=True)

    lane = lax.broadcasted_iota(jnp.int32, (group_size, num_gates), 1).astype(
        jnp.float32
    )
    g1 = jnp.max(raw, axis=-1, keepdims=True)
    i1 = jnp.min(jnp.where(raw >= g1, lane, jnp.float32(1e9)), axis=-1, keepdims=True)
    mask1 = (lane == i1).astype(jnp.float32)
    wo = raw * (1.0 - mask1)
    g2 = jnp.max(wo, axis=-1, keepdims=True)
    i2 = jnp.min(jnp.where(wo >= g2, lane, jnp.float32(1e9)), axis=-1, keepdims=True)
    mask2 = (lane == i2).astype(jnp.float32)

    denom = g1 + g2 + EPS
    g1n = g1 / denom
    g2n = g2 / denom

    proxy_m = jnp.mean(raw, axis=0, keepdims=True)
    dens1 = jnp.mean(mask1, axis=0, keepdims=True)
    partial = jnp.sum(proxy_m * dens1)

    r = lax.broadcasted_iota(jnp.int32, (group_size, group_size), 0)
    c = lax.broadcasted_iota(jnp.int32, (group_size, group_size), 1)
    lt = (r > c).astype(jnp.bfloat16)
    pos1 = jnp.dot(lt, mask1.astype(jnp.bfloat16),
                   preferred_element_type=jnp.float32)
    pos1_tok = jnp.sum(pos1 * mask1, axis=-1, keepdims=True)
    keep1 = (pos1_tok < cap).astype(jnp.float32)
    cnt1 = jnp.sum(mask1 * keep1, axis=0, keepdims=True)
    pos2 = jnp.dot(lt, mask2.astype(jnp.bfloat16),
                   preferred_element_type=jnp.float32) + cnt1
    pos2_tok = jnp.sum(pos2 * mask2, axis=-1, keepdims=True)
    keep2 = (pos2_tok < cap).astype(jnp.float32)

    val1 = g1n * keep1
    val2 = g2n * keep2
    idx1 = jnp.where(keep1 > 0.0, i1 * cap + pos1_tok, jnp.float32(-1.0))
    idx2 = jnp.where(keep2 > 0.0, i2 * cap + pos2_tok, jnp.float32(-1.0))
    losscol = jnp.zeros((group_size, 1), jnp.float32) + partial
    pad = jnp.zeros((group_size, 1), jnp.float32)
    meta_ref[0] = jnp.concatenate(
        [idx1, val1, idx2, val2, losscol, pad, pad, pad], axis=1
    )


def _phase2_body(meta_ref, comb_ref, disp_ref, *, num_cols, ts):
    meta = meta_ref[0]
    i1 = meta[:, 0:1]
    v1 = meta[:, 1:2]
    i2 = meta[:, 2:3]
    v2 = meta[:, 3:4]
    col = lax.broadcasted_iota(jnp.int32, (ts, num_cols), 1).astype(jnp.float32)
    m1 = col == i1
    m2 = col == i2
    comb_ref[0] = jnp.where(m1, v1, 0.0) + jnp.where(m2, v2, 0.0)
    disp_ref[0] = jnp.where(m1, 1.0, 0.0) + jnp.where(m2, 1.0, 0.0)


def kernel(x, W):
    b, s, d = x.shape
    e = W.shape[0]
    cap = max(min(s, int(s * CAPACITY_FACTOR / e)), MIN_CAPACITY)
    nc = e * cap

    meta = pl.pallas_call(
        functools.partial(
            _phase1_body, cap=float(cap), num_gates=e, group_size=s
        ),
        grid=(b,),
        in_specs=[
            pl.BlockSpec((1, s, d), lambda i: (i, 0, 0)),
            pl.BlockSpec((e, d), lambda i: (0, 0)),
        ],
        out_specs=pl.BlockSpec((1, s, 8), lambda i: (i, 0, 0)),
        out_shape=jax.ShapeDtypeStruct((b, s, 8), jnp.float32),
    )(x, W)

    ts = 256
    comb2, disp2 = pl.pallas_call(
        functools.partial(_phase2_body, num_cols=nc, ts=ts),
        grid=(b, s // ts),
        in_specs=[pl.BlockSpec((1, ts, 8), lambda i, j: (i, j, 0))],
        out_specs=[
            pl.BlockSpec((1, ts, nc), lambda i, j: (i, j, 0)),
            pl.BlockSpec((1, ts, nc), lambda i, j: (i, j, 0)),
        ],
        out_shape=[
            jax.ShapeDtypeStruct((b, s, nc), jnp.float32),
            jax.ShapeDtypeStruct((b, s, nc), jnp.float32),
        ],
    )(meta)

    dispatch = disp2.reshape(b, s, e, cap)
    combine = comb2.reshape(b, s, e, cap)
    loss = jnp.sum(meta[:, 0, 4]) * (float(e) / float(b))
    return (dispatch, combine, loss)
